# Initial kernel scaffold; baseline (speedup 1.0000x reference)
import jax
import jax.numpy as jnp
from jax.experimental import pallas as pl


def kernel(edge_index, embed, hyperedge_chunk_sizes, W1, b1, gamma, beta, W2, b2):
    raise NotImplementedError("write your pallas kernel here")


# speedup vs baseline: 2.6202x; 2.6202x over previous
import functools

import jax
import jax.numpy as jnp
from jax import lax
from jax.experimental import pallas as pl
from jax.experimental.pallas import tpu as pltpu
from jax.experimental.pallas import tpu_sc as plsc

D = 256
NCLS = 64
NC, NS = 2, 16
NW = NC * NS
CHUNK = 128


def _node_mm_body(x_ref, wa_ref, wb_ref, b1_ref, pa_ref, pb_ref):
    x = x_ref[...]
    dn = (((1,), (0,)), ((), ()))
    pa_ref[...] = (
        lax.dot_general(x, wa_ref[...], dn, precision=lax.Precision.HIGHEST,
                        preferred_element_type=jnp.float32)
        + b1_ref[...]
    )
    pb_ref[...] = lax.dot_general(
        x, wb_ref[...], dn, precision=lax.Precision.HIGHEST,
        preferred_element_type=jnp.float32)


def _node_tables(embed, wa, wb, b1_row):
    n = embed.shape[0]
    blk = 2000
    grid = n // blk
    return pl.pallas_call(
        _node_mm_body,
        grid=(grid,),
        in_specs=[
            pl.BlockSpec((blk, D), lambda i: (i, 0)),
            pl.BlockSpec((D, D), lambda i: (0, 0)),
            pl.BlockSpec((D, D), lambda i: (0, 0)),
            pl.BlockSpec((1, D), lambda i: (0, 0)),
        ],
        out_specs=[
            pl.BlockSpec((blk, D), lambda i: (i, 0)),
            pl.BlockSpec((blk, D), lambda i: (i, 0)),
        ],
        out_shape=[
            jax.ShapeDtypeStruct((n, D), jnp.float32),
            jax.ShapeDtypeStruct((n, D), jnp.float32),
        ],
    )(embed, wa, wb, b1_row)


def _gather_add(src, snk, pa, pb, e_pad):
    chunks_per_w = e_pad // (NW * CHUNK)
    mesh = plsc.VectorSubcoreMesh(core_axis_name="c", subcore_axis_name="s")

    @functools.partial(
        pl.kernel,
        mesh=mesh,
        out_type=jax.ShapeDtypeStruct((e_pad, D), jnp.float32),
        scratch_types=[
            pltpu.VMEM((CHUNK,), jnp.int32),
            pltpu.VMEM((CHUNK,), jnp.int32),
            pltpu.VMEM((CHUNK, D), jnp.float32),
            pltpu.VMEM((CHUNK, D), jnp.float32),
            pltpu.SemaphoreType.DMA,
        ],
    )
    def body(src_hbm, snk_hbm, pa_hbm, pb_hbm, g_hbm,
             idxa_v, idxb_v, rowsa_v, rowsb_v, sem):
        wid = lax.axis_index("s") * NC + lax.axis_index("c")

        def chunk_body(k, carry):
            base = pl.multiple_of((wid * chunks_per_w + k) * CHUNK, 8)
            pltpu.sync_copy(src_hbm.at[pl.ds(base, CHUNK)], idxa_v)
            pltpu.sync_copy(snk_hbm.at[pl.ds(base, CHUNK)], idxb_v)
            cpa = pltpu.async_copy(pa_hbm.at[idxa_v], rowsa_v, sem)
            cpb = pltpu.async_copy(pb_hbm.at[idxb_v], rowsb_v, sem)
            cpa.wait()
            cpb.wait()

            def add_row(r, c2):
                for c0 in range(D // 16):
                    sl = pl.ds(c0 * 16, 16)
                    rowsa_v[r, sl] = rowsa_v[r, sl] + rowsb_v[r, sl]
                return c2

            lax.fori_loop(0, CHUNK, add_row, 0)
            pltpu.sync_copy(rowsa_v, g_hbm.at[pl.ds(base, CHUNK)])
            return carry

        lax.fori_loop(0, chunks_per_w, chunk_body, 0)

    return body(src, snk, pa, pb)


def _stats_body(g_ref, s_ref, q_ref):
    @pl.when(pl.program_id(0) == 0)
    def _():
        s_ref[...] = jnp.zeros_like(s_ref)
        q_ref[...] = jnp.zeros_like(q_ref)

    h = jnp.maximum(g_ref[...], 0.0)
    s_ref[...] += jnp.sum(h, axis=0, keepdims=True)
    q_ref[...] += jnp.sum(h * h, axis=0, keepdims=True)


def _stats(g, n_edges):
    blk = 2000
    grid = n_edges // blk
    return pl.pallas_call(
        _stats_body,
        grid=(grid,),
        in_specs=[pl.BlockSpec((blk, D), lambda i: (i, 0))],
        out_specs=[
            pl.BlockSpec((1, D), lambda i: (0, 0)),
            pl.BlockSpec((1, D), lambda i: (0, 0)),
        ],
        out_shape=[
            jax.ShapeDtypeStruct((1, D), jnp.float32),
            jax.ShapeDtypeStruct((1, D), jnp.float32),
        ],
    )(g)


def _fc2_body(n_edges, g_ref, s_ref, q_ref, gamma_ref, beta_ref, w2_ref,
              b2_ref, out_ref):
    inv_n = 1.0 / float(n_edges)
    mean = s_ref[...] * inv_n
    var = q_ref[...] * inv_n - mean * mean
    inv = lax.rsqrt(var + 1e-5)
    scale = gamma_ref[...] * inv
    shift = beta_ref[...] - mean * scale

    w2 = w2_ref[...]
    w2e = w2 * scale
    dn = (((1,), (1,)), ((), ()))
    h = jnp.maximum(g_ref[...], 0.0)
    y = lax.dot_general(h, w2e, dn, preferred_element_type=jnp.float32)
    sb = lax.dot_general(shift, w2, dn, preferred_element_type=jnp.float32)
    out_ref[...] = y + sb + b2_ref[...]


def _fc2(g, s, q, gamma_row, beta_row, w2, b2_row, n_edges):
    blk = 2000
    grid = n_edges // blk
    return pl.pallas_call(
        functools.partial(_fc2_body, n_edges),
        grid=(grid,),
        in_specs=[
            pl.BlockSpec((blk, D), lambda i: (i, 0)),
            pl.BlockSpec((1, D), lambda i: (0, 0)),
            pl.BlockSpec((1, D), lambda i: (0, 0)),
            pl.BlockSpec((1, D), lambda i: (0, 0)),
            pl.BlockSpec((1, D), lambda i: (0, 0)),
            pl.BlockSpec((NCLS, D), lambda i: (0, 0)),
            pl.BlockSpec((1, NCLS), lambda i: (0, 0)),
        ],
        out_specs=pl.BlockSpec((blk, NCLS), lambda i: (i, 0)),
        out_shape=jax.ShapeDtypeStruct((n_edges, NCLS), jnp.float32),
    )(g, s, q, gamma_row, beta_row, w2, b2_row)


def kernel(edge_index, embed, hyperedge_chunk_sizes, W1, b1, gamma, beta,
           W2, b2):
    e = edge_index.shape[1]
    src = edge_index[0]
    snk = edge_index[1]

    per_w = -(-e // (NW * CHUNK)) * CHUNK
    e_pad = per_w * NW
    if e_pad != e:
        pad = e_pad - e
        src = jnp.concatenate([src, jnp.zeros((pad,), jnp.int32)])
        snk = jnp.concatenate([snk, jnp.zeros((pad,), jnp.int32)])

    wa = W1[:, :D].T
    wb = W1[:, D:].T
    pa, pb = _node_tables(embed, wa, wb, b1.reshape(1, D))

    g = _gather_add(src, snk, pa, pb, e_pad)

    s, q = _stats(g, e)
    out = _fc2(g, s, q, gamma.reshape(1, D), beta.reshape(1, D), W2,
               b2.reshape(1, NCLS), e)
    return out


# speedup vs baseline: 4.3019x; 1.6418x over previous
import functools

import jax
import jax.numpy as jnp
from jax import lax
from jax.experimental import pallas as pl
from jax.experimental.pallas import tpu as pltpu
from jax.experimental.pallas import tpu_sc as plsc

D = 256
NCLS = 64
NC, NS = 2, 16
NW = NC * NS
CHUNK = 120


def _node_mm_body(x_ref, wa_ref, wb_ref, b1_ref, pa_ref, pb_ref):
    x = x_ref[...]
    dn = (((1,), (0,)), ((), ()))
    pa_ref[...] = (
        lax.dot_general(x, wa_ref[...], dn, precision=lax.Precision.HIGHEST,
                        preferred_element_type=jnp.float32)
        + b1_ref[...]
    )
    pb_ref[...] = lax.dot_general(
        x, wb_ref[...], dn, precision=lax.Precision.HIGHEST,
        preferred_element_type=jnp.float32)


def _node_tables(embed, wa, wb, b1_row):
    n = embed.shape[0]
    blk = 2000
    grid = n // blk
    return pl.pallas_call(
        _node_mm_body,
        grid=(grid,),
        in_specs=[
            pl.BlockSpec((blk, D), lambda i: (i, 0)),
            pl.BlockSpec((D, D), lambda i: (0, 0)),
            pl.BlockSpec((D, D), lambda i: (0, 0)),
            pl.BlockSpec((1, D), lambda i: (0, 0)),
        ],
        out_specs=[
            pl.BlockSpec((blk, D), lambda i: (i, 0)),
            pl.BlockSpec((blk, D), lambda i: (i, 0)),
        ],
        out_shape=[
            jax.ShapeDtypeStruct((n, D), jnp.float32),
            jax.ShapeDtypeStruct((n, D), jnp.float32),
        ],
    )(embed, wa, wb, b1_row)


def _gather_add(src, snk, pa, pb, e_pad):
    chunks_per_w = e_pad // (NW * CHUNK)
    assert chunks_per_w % 2 == 0
    mesh = plsc.VectorSubcoreMesh(core_axis_name="c", subcore_axis_name="s")

    buf_types = [
        pltpu.VMEM((CHUNK,), jnp.int32),
        pltpu.VMEM((CHUNK,), jnp.int32),
        pltpu.VMEM((CHUNK, D), jnp.float32),
        pltpu.VMEM((CHUNK, D), jnp.float32),
        pltpu.SemaphoreType.DMA,
    ]

    @functools.partial(
        pl.kernel,
        mesh=mesh,
        out_type=jax.ShapeDtypeStruct((e_pad, D), jnp.float32),
        scratch_types=buf_types + buf_types,
    )
    def body(src_hbm, snk_hbm, pa_hbm, pb_hbm, g_hbm,
             ia0, ib0, ra0, rb0, sem0, ia1, ib1, ra1, rb1, sem1):
        wid = lax.axis_index("s") * NC + lax.axis_index("c")
        base0 = wid * chunks_per_w * CHUNK

        def fire(k, ia, ib, ra, rb, sem):
            base = pl.multiple_of(base0 + k * CHUNK, 8)
            pltpu.sync_copy(src_hbm.at[pl.ds(base, CHUNK)], ia)
            pltpu.sync_copy(snk_hbm.at[pl.ds(base, CHUNK)], ib)
            pltpu.async_copy(pa_hbm.at[ia], ra, sem)
            pltpu.async_copy(pb_hbm.at[ib], rb, sem)

        def drain_add_store(k, ia, ib, ra, rb, sem):
            pltpu.make_async_copy(pa_hbm.at[ia], ra, sem).wait()
            pltpu.make_async_copy(pb_hbm.at[ib], rb, sem).wait()

            def add_row(r, c2):
                for c0 in range(D // 16):
                    sl = pl.ds(c0 * 16, 16)
                    ra[r, sl] = ra[r, sl] + rb[r, sl]
                return c2

            lax.fori_loop(0, CHUNK, add_row, 0)
            base = pl.multiple_of(base0 + k * CHUNK, 8)
            pltpu.sync_copy(ra, g_hbm.at[pl.ds(base, CHUNK)])

        fire(0, ia0, ib0, ra0, rb0, sem0)

        def pair_body(j, carry):
            k0 = 2 * j
            fire(k0 + 1, ia1, ib1, ra1, rb1, sem1)
            drain_add_store(k0, ia0, ib0, ra0, rb0, sem0)

            @pl.when(j < chunks_per_w // 2 - 1)
            def _():
                fire(k0 + 2, ia0, ib0, ra0, rb0, sem0)

            drain_add_store(k0 + 1, ia1, ib1, ra1, rb1, sem1)
            return carry

        lax.fori_loop(0, chunks_per_w // 2, pair_body, 0)

    return body(src, snk, pa, pb)


def _stats_body(g_ref, s_ref, q_ref):
    @pl.when(pl.program_id(0) == 0)
    def _():
        s_ref[...] = jnp.zeros_like(s_ref)
        q_ref[...] = jnp.zeros_like(q_ref)

    h = jnp.maximum(g_ref[...], 0.0)
    s_ref[...] += jnp.sum(h, axis=0, keepdims=True)
    q_ref[...] += jnp.sum(h * h, axis=0, keepdims=True)


def _stats(g, n_edges):
    blk = 2000
    grid = n_edges // blk
    return pl.pallas_call(
        _stats_body,
        grid=(grid,),
        in_specs=[pl.BlockSpec((blk, D), lambda i: (i, 0))],
        out_specs=[
            pl.BlockSpec((1, D), lambda i: (0, 0)),
            pl.BlockSpec((1, D), lambda i: (0, 0)),
        ],
        out_shape=[
            jax.ShapeDtypeStruct((1, D), jnp.float32),
            jax.ShapeDtypeStruct((1, D), jnp.float32),
        ],
    )(g)


def _fc2_body(n_edges, g_ref, s_ref, q_ref, gamma_ref, beta_ref, w2_ref,
              b2_ref, out_ref):
    inv_n = 1.0 / float(n_edges)
    mean = s_ref[...] * inv_n
    var = q_ref[...] * inv_n - mean * mean
    inv = lax.rsqrt(var + 1e-5)
    scale = gamma_ref[...] * inv
    shift = beta_ref[...] - mean * scale

    w2 = w2_ref[...]
    w2e = w2 * scale
    dn = (((1,), (1,)), ((), ()))
    h = jnp.maximum(g_ref[...], 0.0)
    y = lax.dot_general(h, w2e, dn, preferred_element_type=jnp.float32)
    sb = lax.dot_general(shift, w2, dn, preferred_element_type=jnp.float32)
    out_ref[...] = y + sb + b2_ref[...]


def _fc2(g, s, q, gamma_row, beta_row, w2, b2_row, n_edges):
    blk = 2000
    grid = n_edges // blk
    return pl.pallas_call(
        functools.partial(_fc2_body, n_edges),
        grid=(grid,),
        in_specs=[
            pl.BlockSpec((blk, D), lambda i: (i, 0)),
            pl.BlockSpec((1, D), lambda i: (0, 0)),
            pl.BlockSpec((1, D), lambda i: (0, 0)),
            pl.BlockSpec((1, D), lambda i: (0, 0)),
            pl.BlockSpec((1, D), lambda i: (0, 0)),
            pl.BlockSpec((NCLS, D), lambda i: (0, 0)),
            pl.BlockSpec((1, NCLS), lambda i: (0, 0)),
        ],
        out_specs=pl.BlockSpec((blk, NCLS), lambda i: (i, 0)),
        out_shape=jax.ShapeDtypeStruct((n_edges, NCLS), jnp.float32),
    )(g, s, q, gamma_row, beta_row, w2, b2_row)


def kernel(edge_index, embed, hyperedge_chunk_sizes, W1, b1, gamma, beta,
           W2, b2):
    e = edge_index.shape[1]
    src = edge_index[0]
    snk = edge_index[1]

    per_w = -(-e // (NW * CHUNK)) * CHUNK
    e_pad = per_w * NW
    if e_pad != e:
        pad = e_pad - e
        src = jnp.concatenate([src, jnp.zeros((pad,), jnp.int32)])
        snk = jnp.concatenate([snk, jnp.zeros((pad,), jnp.int32)])

    wa = W1[:, :D].T
    wb = W1[:, D:].T
    pa, pb = _node_tables(embed, wa, wb, b1.reshape(1, D))

    g = _gather_add(src, snk, pa, pb, e_pad)

    s, q = _stats(g, e)
    out = _fc2(g, s, q, gamma.reshape(1, D), beta.reshape(1, D), W2,
               b2.reshape(1, NCLS), e)
    return out


# speedup vs baseline: 4.3685x; 1.0155x over previous
import functools

import jax
import jax.numpy as jnp
from jax import lax
from jax.experimental import pallas as pl
from jax.experimental.pallas import tpu as pltpu
from jax.experimental.pallas import tpu_sc as plsc

D = 256
NCLS = 64
NC, NS = 2, 16
NW = NC * NS
CHUNK = 80


def _node_mm_body(x_ref, wa_ref, wb_ref, b1_ref, pa_ref, pb_ref):
    x = x_ref[...]
    dn = (((1,), (0,)), ((), ()))
    pa_ref[...] = (
        lax.dot_general(x, wa_ref[...], dn, precision=lax.Precision.HIGHEST,
                        preferred_element_type=jnp.float32)
        + b1_ref[...]
    )
    pb_ref[...] = lax.dot_general(
        x, wb_ref[...], dn, precision=lax.Precision.HIGHEST,
        preferred_element_type=jnp.float32)


def _node_tables(embed, wa, wb, b1_row):
    n = embed.shape[0]
    blk = 2000
    grid = n // blk
    return pl.pallas_call(
        _node_mm_body,
        grid=(grid,),
        in_specs=[
            pl.BlockSpec((blk, D), lambda i: (i, 0)),
            pl.BlockSpec((D, D), lambda i: (0, 0)),
            pl.BlockSpec((D, D), lambda i: (0, 0)),
            pl.BlockSpec((1, D), lambda i: (0, 0)),
        ],
        out_specs=[
            pl.BlockSpec((blk, D), lambda i: (i, 0)),
            pl.BlockSpec((blk, D), lambda i: (i, 0)),
        ],
        out_shape=[
            jax.ShapeDtypeStruct((n, D), jnp.float32),
            jax.ShapeDtypeStruct((n, D), jnp.float32),
        ],
    )(embed, wa, wb, b1_row)


def _gather_add(src, snk, pa, pb, e_pad):
    chunks_per_w = e_pad // (NW * CHUNK)
    assert chunks_per_w % 3 == 0
    mesh = plsc.VectorSubcoreMesh(core_axis_name="c", subcore_axis_name="s")

    buf_types = [
        pltpu.VMEM((CHUNK,), jnp.int32),
        pltpu.VMEM((CHUNK,), jnp.int32),
        pltpu.VMEM((CHUNK, D), jnp.float32),
        pltpu.VMEM((CHUNK, D), jnp.float32),
        pltpu.SemaphoreType.DMA,
        pltpu.SemaphoreType.DMA,
    ]

    @functools.partial(
        pl.kernel,
        mesh=mesh,
        out_type=jax.ShapeDtypeStruct((e_pad, D), jnp.float32),
        scratch_types=buf_types * 3,
    )
    def body(src_hbm, snk_hbm, pa_hbm, pb_hbm, g_hbm, *bufs):
        wid = lax.axis_index("s") * NC + lax.axis_index("c")
        base0 = wid * chunks_per_w * CHUNK
        B = [bufs[6 * i: 6 * (i + 1)] for i in range(3)]

        def ebase(k):
            return pl.multiple_of(base0 + k * CHUNK, 8)

        def fire(k, buf):
            ia, ib, ra, rb, gsem, _ = buf
            base = ebase(k)
            pltpu.sync_copy(src_hbm.at[pl.ds(base, CHUNK)], ia)
            pltpu.sync_copy(snk_hbm.at[pl.ds(base, CHUNK)], ib)
            pltpu.async_copy(pa_hbm.at[ia], ra, gsem)
            pltpu.async_copy(pb_hbm.at[ib], rb, gsem)

        def wait_store(buf):
            _, _, ra, _, _, ssem = buf
            pltpu.make_async_copy(ra, g_hbm.at[pl.ds(base0, CHUNK)],
                                  ssem).wait()

        def drain_add_store(k, buf):
            ia, ib, ra, rb, gsem, ssem = buf
            pltpu.make_async_copy(pa_hbm.at[ia], ra, gsem).wait()
            pltpu.make_async_copy(pb_hbm.at[ib], rb, gsem).wait()

            def add_row(r, c2):
                for c0 in range(D // 16):
                    sl = pl.ds(c0 * 16, 16)
                    ra[r, sl] = ra[r, sl] + rb[r, sl]
                return c2

            lax.fori_loop(0, CHUNK, add_row, 0)
            pltpu.async_copy(ra, g_hbm.at[pl.ds(ebase(k), CHUNK)], ssem)

        fire(0, B[0])
        fire(1, B[1])
        pltpu.async_copy(B[2][2], g_hbm.at[pl.ds(ebase(2), CHUNK)], B[2][5])

        def tri_body(j, carry):
            k0 = 3 * j
            drain_add_store(k0, B[0])
            wait_store(B[2])

            @pl.when(k0 + 2 < chunks_per_w)
            def _():
                fire(k0 + 2, B[2])

            drain_add_store(k0 + 1, B[1])
            wait_store(B[0])

            @pl.when(k0 + 3 < chunks_per_w)
            def _():
                fire(k0 + 3, B[0])

            drain_add_store(k0 + 2, B[2])
            wait_store(B[1])

            @pl.when(k0 + 4 < chunks_per_w)
            def _():
                fire(k0 + 4, B[1])

            return carry

        lax.fori_loop(0, chunks_per_w // 3, tri_body, 0)
        wait_store(B[2])

    return body(src, snk, pa, pb)


def _stats_body(g_ref, s_ref, q_ref):
    @pl.when(pl.program_id(0) == 0)
    def _():
        s_ref[...] = jnp.zeros_like(s_ref)
        q_ref[...] = jnp.zeros_like(q_ref)

    h = jnp.maximum(g_ref[...], 0.0)
    s_ref[...] += jnp.sum(h, axis=0, keepdims=True)
    q_ref[...] += jnp.sum(h * h, axis=0, keepdims=True)


def _stats(g, n_edges):
    blk = 2000
    grid = n_edges // blk
    return pl.pallas_call(
        _stats_body,
        grid=(grid,),
        in_specs=[pl.BlockSpec((blk, D), lambda i: (i, 0))],
        out_specs=[
            pl.BlockSpec((1, D), lambda i: (0, 0)),
            pl.BlockSpec((1, D), lambda i: (0, 0)),
        ],
        out_shape=[
            jax.ShapeDtypeStruct((1, D), jnp.float32),
            jax.ShapeDtypeStruct((1, D), jnp.float32),
        ],
    )(g)


def _fc2_body(n_edges, g_ref, s_ref, q_ref, gamma_ref, beta_ref, w2_ref,
              b2_ref, out_ref):
    inv_n = 1.0 / float(n_edges)
    mean = s_ref[...] * inv_n
    var = q_ref[...] * inv_n - mean * mean
    inv = lax.rsqrt(var + 1e-5)
    scale = gamma_ref[...] * inv
    shift = beta_ref[...] - mean * scale

    w2 = w2_ref[...]
    w2e = w2 * scale
    dn = (((1,), (1,)), ((), ()))
    h = jnp.maximum(g_ref[...], 0.0)
    y = lax.dot_general(h, w2e, dn, preferred_element_type=jnp.float32)
    sb = lax.dot_general(shift, w2, dn, preferred_element_type=jnp.float32)
    out_ref[...] = y + sb + b2_ref[...]


def _fc2(g, s, q, gamma_row, beta_row, w2, b2_row, n_edges):
    blk = 2000
    grid = n_edges // blk
    return pl.pallas_call(
        functools.partial(_fc2_body, n_edges),
        grid=(grid,),
        in_specs=[
            pl.BlockSpec((blk, D), lambda i: (i, 0)),
            pl.BlockSpec((1, D), lambda i: (0, 0)),
            pl.BlockSpec((1, D), lambda i: (0, 0)),
            pl.BlockSpec((1, D), lambda i: (0, 0)),
            pl.BlockSpec((1, D), lambda i: (0, 0)),
            pl.BlockSpec((NCLS, D), lambda i: (0, 0)),
            pl.BlockSpec((1, NCLS), lambda i: (0, 0)),
        ],
        out_specs=pl.BlockSpec((blk, NCLS), lambda i: (i, 0)),
        out_shape=jax.ShapeDtypeStruct((n_edges, NCLS), jnp.float32),
    )(g, s, q, gamma_row, beta_row, w2, b2_row)


def kernel(edge_index, embed, hyperedge_chunk_sizes, W1, b1, gamma, beta,
           W2, b2):
    e = edge_index.shape[1]
    src = edge_index[0]
    snk = edge_index[1]

    per_w = -(-e // (NW * CHUNK)) * CHUNK
    e_pad = per_w * NW
    if e_pad != e:
        pad = e_pad - e
        src = jnp.concatenate([src, jnp.zeros((pad,), jnp.int32)])
        snk = jnp.concatenate([snk, jnp.zeros((pad,), jnp.int32)])

    wa = W1[:, :D].T
    wb = W1[:, D:].T
    pa, pb = _node_tables(embed, wa, wb, b1.reshape(1, D))

    g = _gather_add(src, snk, pa, pb, e_pad)

    s, q = _stats(g, e)
    out = _fc2(g, s, q, gamma.reshape(1, D), beta.reshape(1, D), W2,
               b2.reshape(1, NCLS), e)
    return out


# speedup vs baseline: 4.8633x; 1.1133x over previous
import functools

import jax
import jax.numpy as jnp
from jax import lax
from jax.experimental import pallas as pl
from jax.experimental.pallas import tpu as pltpu
from jax.experimental.pallas import tpu_sc as plsc

D = 256
NCLS = 64
NC, NS = 2, 16
NW = NC * NS
CHUNK = 80


def _node_mm_body(x_ref, wa_ref, wb_ref, b1_ref, pa_ref, pb_ref):
    x = x_ref[...]
    dn = (((1,), (0,)), ((), ()))
    pa_ref[...] = (
        lax.dot_general(x, wa_ref[...], dn, precision=lax.Precision.HIGHEST,
                        preferred_element_type=jnp.float32)
        + b1_ref[...]
    )
    pb_ref[...] = lax.dot_general(
        x, wb_ref[...], dn, precision=lax.Precision.HIGHEST,
        preferred_element_type=jnp.float32)


def _node_tables(embed, wa, wb, b1_row):
    n = embed.shape[0]
    blk = 2000
    grid = n // blk
    return pl.pallas_call(
        _node_mm_body,
        grid=(grid,),
        in_specs=[
            pl.BlockSpec((blk, D), lambda i: (i, 0)),
            pl.BlockSpec((D, D), lambda i: (0, 0)),
            pl.BlockSpec((D, D), lambda i: (0, 0)),
            pl.BlockSpec((1, D), lambda i: (0, 0)),
        ],
        out_specs=[
            pl.BlockSpec((blk, D), lambda i: (i, 0)),
            pl.BlockSpec((blk, D), lambda i: (i, 0)),
        ],
        out_shape=[
            jax.ShapeDtypeStruct((n, D), jnp.float32),
            jax.ShapeDtypeStruct((n, D), jnp.float32),
        ],
    )(embed, wa, wb, b1_row)


def _gather_add(src, snk, pa, pb, e_pad):
    chunks_per_w = e_pad // (NW * CHUNK)
    assert chunks_per_w % 3 == 0
    mesh = plsc.VectorSubcoreMesh(core_axis_name="c", subcore_axis_name="s")

    buf_types = [
        pltpu.VMEM((CHUNK,), jnp.int32),
        pltpu.VMEM((CHUNK,), jnp.int32),
        pltpu.VMEM((CHUNK, D), jnp.float32),
        pltpu.VMEM((CHUNK, D), jnp.float32),
        pltpu.SemaphoreType.DMA,
        pltpu.SemaphoreType.DMA,
    ]

    @functools.partial(
        pl.kernel,
        mesh=mesh,
        out_type=[
            jax.ShapeDtypeStruct((e_pad, D), jnp.float32),
            jax.ShapeDtypeStruct((NW, 2, D), jnp.float32),
        ],
        scratch_types=[pltpu.VMEM((2, D), jnp.float32)] + buf_types * 3,
    )
    def body(src_hbm, snk_hbm, pa_hbm, pb_hbm, g_hbm, st_hbm, st_v, *bufs):
        wid = lax.axis_index("s") * NC + lax.axis_index("c")
        base0 = wid * chunks_per_w * CHUNK
        B = [bufs[6 * i: 6 * (i + 1)] for i in range(3)]

        def ebase(k):
            return pl.multiple_of(base0 + k * CHUNK, 8)

        def fire(k, buf):
            ia, ib, ra, rb, gsem, _ = buf
            base = ebase(k)
            pltpu.sync_copy(src_hbm.at[pl.ds(base, CHUNK)], ia)
            pltpu.sync_copy(snk_hbm.at[pl.ds(base, CHUNK)], ib)
            pltpu.async_copy(pa_hbm.at[ia], ra, gsem)
            pltpu.async_copy(pb_hbm.at[ib], rb, gsem)

        def wait_store(buf):
            ra, ssem = buf[2], buf[5]
            pltpu.make_async_copy(ra, g_hbm.at[pl.ds(base0, CHUNK)],
                                  ssem).wait()

        def drain_add_store(k, buf, acc):
            ia, ib, ra, rb, gsem, ssem = buf
            pltpu.make_async_copy(pa_hbm.at[ia], ra, gsem).wait()
            pltpu.make_async_copy(pb_hbm.at[ib], rb, gsem).wait()

            def add_row(r, a):
                out = []
                for c0 in range(D // 16):
                    sl = pl.ds(16 * c0, 16)
                    g = ra[r, sl] + rb[r, sl]
                    ra[r, sl] = g
                    h = jnp.maximum(g, 0.0)
                    out.append(a[2 * c0] + h)
                    out.append(a[2 * c0 + 1] + h * h)
                return tuple(out)

            acc = lax.fori_loop(0, CHUNK, add_row, acc)
            pltpu.async_copy(ra, g_hbm.at[pl.ds(ebase(k), CHUNK)], ssem)
            return acc

        fire(0, B[0])
        fire(1, B[1])
        pltpu.async_copy(B[2][2], g_hbm.at[pl.ds(ebase(2), CHUNK)],
                         B[2][5])

        zero = jnp.zeros((16,), jnp.float32)
        acc0 = (zero,) * (2 * (D // 16))

        def tri_body(j, acc):
            k0 = 3 * j
            acc = drain_add_store(k0, B[0], acc)
            wait_store(B[2])

            @pl.when(k0 + 2 < chunks_per_w)
            def _():
                fire(k0 + 2, B[2])

            acc = drain_add_store(k0 + 1, B[1], acc)
            wait_store(B[0])

            @pl.when(k0 + 3 < chunks_per_w)
            def _():
                fire(k0 + 3, B[0])

            acc = drain_add_store(k0 + 2, B[2], acc)
            wait_store(B[1])

            @pl.when(k0 + 4 < chunks_per_w)
            def _():
                fire(k0 + 4, B[1])

            return acc

        acc = lax.fori_loop(0, chunks_per_w // 3, tri_body, acc0)
        wait_store(B[2])
        for c0 in range(D // 16):
            st_v[0, pl.ds(16 * c0, 16)] = acc[2 * c0]
            st_v[1, pl.ds(16 * c0, 16)] = acc[2 * c0 + 1]
        pltpu.sync_copy(st_v, st_hbm.at[wid])

    return body(src, snk, pa, pb)


def _fc2_body(n_edges, n_pad, g_ref, st_ref, pa0_ref, pb0_ref, gamma_ref,
              beta_ref, w2_ref, b2_ref, out_ref):
    st = st_ref[...]
    hp = jnp.maximum(pa0_ref[...] + pb0_ref[...], 0.0)
    s = jnp.sum(st[:, 0, :], axis=0, keepdims=True) - n_pad * hp
    q = jnp.sum(st[:, 1, :], axis=0, keepdims=True) - n_pad * hp * hp
    inv_n = 1.0 / float(n_edges)
    mean = s * inv_n
    var = q * inv_n - mean * mean
    inv = lax.rsqrt(var + 1e-5)
    scale = gamma_ref[...] * inv
    shift = beta_ref[...] - mean * scale

    w2 = w2_ref[...]
    w2e = w2 * scale
    dn = (((1,), (1,)), ((), ()))
    h = jnp.maximum(g_ref[...], 0.0)
    y = lax.dot_general(h, w2e, dn, preferred_element_type=jnp.float32)
    sb = lax.dot_general(shift, w2, dn, preferred_element_type=jnp.float32)
    out_ref[...] = y + sb + b2_ref[...]


def _fc2(g, st, pa0, pb0, gamma_row, beta_row, w2, b2_row, n_edges, n_pad):
    blk = 2000
    grid = n_edges // blk
    return pl.pallas_call(
        functools.partial(_fc2_body, n_edges, float(n_pad)),
        grid=(grid,),
        in_specs=[
            pl.BlockSpec((blk, D), lambda i: (i, 0)),
            pl.BlockSpec((NW, 2, D), lambda i: (0, 0, 0)),
            pl.BlockSpec((1, D), lambda i: (0, 0)),
            pl.BlockSpec((1, D), lambda i: (0, 0)),
            pl.BlockSpec((1, D), lambda i: (0, 0)),
            pl.BlockSpec((1, D), lambda i: (0, 0)),
            pl.BlockSpec((NCLS, D), lambda i: (0, 0)),
            pl.BlockSpec((1, NCLS), lambda i: (0, 0)),
        ],
        out_specs=pl.BlockSpec((blk, NCLS), lambda i: (i, 0)),
        out_shape=jax.ShapeDtypeStruct((n_edges, NCLS), jnp.float32),
    )(g, st, pa0, pb0, gamma_row, beta_row, w2, b2_row)


def kernel(edge_index, embed, hyperedge_chunk_sizes, W1, b1, gamma, beta,
           W2, b2):
    e = edge_index.shape[1]
    src = edge_index[0]
    snk = edge_index[1]

    n_chunks = -(-e // (NW * CHUNK))
    n_chunks += (-n_chunks) % 3
    e_pad = n_chunks * CHUNK * NW
    if e_pad != e:
        pad = e_pad - e
        src = jnp.concatenate([src, jnp.zeros((pad,), jnp.int32)])
        snk = jnp.concatenate([snk, jnp.zeros((pad,), jnp.int32)])

    wa = W1[:, :D].T
    wb = W1[:, D:].T
    pa, pb = _node_tables(embed, wa, wb, b1.reshape(1, D))

    g, st = _gather_add(src, snk, pa, pb, e_pad)

    out = _fc2(g, st, pa[0:1], pb[0:1], gamma.reshape(1, D),
               beta.reshape(1, D), W2, b2.reshape(1, NCLS), e, e_pad - e)
    return out


# speedup vs baseline: 5.2020x; 1.0696x over previous
import functools

import jax
import jax.numpy as jnp
from jax import lax
from jax.experimental import pallas as pl
from jax.experimental.pallas import tpu as pltpu
from jax.experimental.pallas import tpu_sc as plsc

D = 256
NCLS = 64
NC, NS = 2, 16
NW = NC * NS
CHUNK = 80


def _node_mm_body(x_ref, wa_ref, wb_ref, b1_ref, pa_ref, pb_ref):
    x = x_ref[...]
    dn = (((1,), (0,)), ((), ()))
    pa_ref[...] = (
        lax.dot_general(x, wa_ref[...], dn, precision=lax.Precision.HIGHEST,
                        preferred_element_type=jnp.float32)
        + b1_ref[...]
    )
    pb_ref[...] = lax.dot_general(
        x, wb_ref[...], dn, precision=lax.Precision.HIGHEST,
        preferred_element_type=jnp.float32)


def _node_tables(embed, wa, wb, b1_row):
    n = embed.shape[0]
    blk = 2000
    grid = n // blk
    return pl.pallas_call(
        _node_mm_body,
        grid=(grid,),
        in_specs=[
            pl.BlockSpec((blk, D), lambda i: (i, 0)),
            pl.BlockSpec((D, D), lambda i: (0, 0)),
            pl.BlockSpec((D, D), lambda i: (0, 0)),
            pl.BlockSpec((1, D), lambda i: (0, 0)),
        ],
        out_specs=[
            pl.BlockSpec((blk, D), lambda i: (i, 0)),
            pl.BlockSpec((blk, D), lambda i: (i, 0)),
        ],
        out_shape=[
            jax.ShapeDtypeStruct((n, D), jnp.float32),
            jax.ShapeDtypeStruct((n, D), jnp.float32),
        ],
    )(embed, wa, wb, b1_row)


def _gather_add(src, snk, pa, pb, e_pad):
    chunks_per_w = e_pad // (NW * CHUNK)
    assert chunks_per_w % 3 == 0
    mesh = plsc.VectorSubcoreMesh(core_axis_name="c", subcore_axis_name="s")

    buf_types = [
        pltpu.VMEM((CHUNK,), jnp.int32),
        pltpu.VMEM((CHUNK,), jnp.int32),
        pltpu.VMEM((CHUNK, D), jnp.float32),
        pltpu.VMEM((CHUNK, D), jnp.float32),
        pltpu.SemaphoreType.DMA,
        pltpu.SemaphoreType.DMA,
        pltpu.SemaphoreType.DMA,
    ]

    @functools.partial(
        pl.kernel,
        mesh=mesh,
        out_type=[
            jax.ShapeDtypeStruct((e_pad, D), jnp.float32),
            jax.ShapeDtypeStruct((NW, 2, D), jnp.float32),
        ],
        scratch_types=[pltpu.VMEM((2, D), jnp.float32)] + buf_types * 3,
    )
    def body(src_hbm, snk_hbm, pa_hbm, pb_hbm, g_hbm, st_hbm, st_v, *bufs):
        wid = lax.axis_index("s") * NC + lax.axis_index("c")
        base0 = wid * chunks_per_w * CHUNK
        B = [bufs[7 * i: 7 * (i + 1)] for i in range(3)]

        def ebase(k):
            return pl.multiple_of(base0 + k * CHUNK, 8)

        def fire_idx(k, buf):
            ia, ib, isem = buf[0], buf[1], buf[6]
            base = ebase(k)
            pltpu.async_copy(src_hbm.at[pl.ds(base, CHUNK)], ia, isem)
            pltpu.async_copy(snk_hbm.at[pl.ds(base, CHUNK)], ib, isem)

        def fire_gather(buf):
            ia, ib, ra, rb, gsem, isem = (buf[0], buf[1], buf[2], buf[3],
                                          buf[4], buf[6])
            pltpu.make_async_copy(src_hbm.at[pl.ds(base0, CHUNK)], ia,
                                  isem).wait()
            pltpu.make_async_copy(snk_hbm.at[pl.ds(base0, CHUNK)], ib,
                                  isem).wait()
            pltpu.async_copy(pa_hbm.at[ia], ra, gsem)
            pltpu.async_copy(pb_hbm.at[ib], rb, gsem)

        def wait_store(buf):
            ra, ssem = buf[2], buf[5]
            pltpu.make_async_copy(ra, g_hbm.at[pl.ds(base0, CHUNK)],
                                  ssem).wait()

        def drain_add_store(k, buf, acc):
            ia, ib, ra, rb, gsem, ssem, _ = buf
            pltpu.make_async_copy(pa_hbm.at[ia], ra, gsem).wait()
            pltpu.make_async_copy(pb_hbm.at[ib], rb, gsem).wait()

            def add_row(r, a):
                out = []
                for c0 in range(D // 16):
                    sl = pl.ds(16 * c0, 16)
                    g = ra[r, sl] + rb[r, sl]
                    ra[r, sl] = g
                    h = jnp.maximum(g, 0.0)
                    out.append(a[2 * c0] + h)
                    out.append(a[2 * c0 + 1] + h * h)
                return tuple(out)

            acc = lax.fori_loop(0, CHUNK, add_row, acc)
            pltpu.async_copy(ra, g_hbm.at[pl.ds(ebase(k), CHUNK)], ssem)
            return acc

        fire_idx(0, B[0])
        fire_idx(1, B[1])
        fire_idx(2, B[2])
        fire_gather(B[0])
        fire_gather(B[1])
        pltpu.async_copy(B[2][2], g_hbm.at[pl.ds(ebase(2), CHUNK)],
                         B[2][5])

        zero = jnp.zeros((16,), jnp.float32)
        acc0 = (zero,) * (2 * (D // 16))

        def tri_body(j, acc):
            k0 = 3 * j
            acc = drain_add_store(k0, B[0], acc)
            wait_store(B[2])

            @pl.when(k0 + 2 < chunks_per_w)
            def _():
                fire_gather(B[2])

            @pl.when(k0 + 3 < chunks_per_w)
            def _():
                fire_idx(k0 + 3, B[0])

            acc = drain_add_store(k0 + 1, B[1], acc)
            wait_store(B[0])

            @pl.when(k0 + 3 < chunks_per_w)
            def _():
                fire_gather(B[0])

            @pl.when(k0 + 4 < chunks_per_w)
            def _():
                fire_idx(k0 + 4, B[1])

            acc = drain_add_store(k0 + 2, B[2], acc)
            wait_store(B[1])

            @pl.when(k0 + 4 < chunks_per_w)
            def _():
                fire_gather(B[1])

            @pl.when(k0 + 5 < chunks_per_w)
            def _():
                fire_idx(k0 + 5, B[2])

            return acc

        acc = lax.fori_loop(0, chunks_per_w // 3, tri_body, acc0)
        wait_store(B[2])
        for c0 in range(D // 16):
            st_v[0, pl.ds(16 * c0, 16)] = acc[2 * c0]
            st_v[1, pl.ds(16 * c0, 16)] = acc[2 * c0 + 1]
        pltpu.sync_copy(st_v, st_hbm.at[wid])

    return body(src, snk, pa, pb)


def _fc2_body(n_edges, n_pad, g_ref, st_ref, pa0_ref, pb0_ref, gamma_ref,
              beta_ref, w2_ref, b2_ref, out_ref):
    st = st_ref[...]
    hp = jnp.maximum(pa0_ref[...] + pb0_ref[...], 0.0)
    s = jnp.sum(st[:, 0, :], axis=0, keepdims=True) - n_pad * hp
    q = jnp.sum(st[:, 1, :], axis=0, keepdims=True) - n_pad * hp * hp
    inv_n = 1.0 / float(n_edges)
    mean = s * inv_n
    var = q * inv_n - mean * mean
    inv = lax.rsqrt(var + 1e-5)
    scale = gamma_ref[...] * inv
    shift = beta_ref[...] - mean * scale

    w2 = w2_ref[...]
    w2e = w2 * scale
    dn = (((1,), (1,)), ((), ()))
    h = jnp.maximum(g_ref[...], 0.0)
    y = lax.dot_general(h, w2e, dn, preferred_element_type=jnp.float32)
    sb = lax.dot_general(shift, w2, dn, preferred_element_type=jnp.float32)
    out_ref[...] = y + sb + b2_ref[...]


def _fc2(g, st, pa0, pb0, gamma_row, beta_row, w2, b2_row, n_edges, n_pad):
    blk = 2000
    grid = n_edges // blk
    return pl.pallas_call(
        functools.partial(_fc2_body, n_edges, float(n_pad)),
        grid=(grid,),
        in_specs=[
            pl.BlockSpec((blk, D), lambda i: (i, 0)),
            pl.BlockSpec((NW, 2, D), lambda i: (0, 0, 0)),
            pl.BlockSpec((1, D), lambda i: (0, 0)),
            pl.BlockSpec((1, D), lambda i: (0, 0)),
            pl.BlockSpec((1, D), lambda i: (0, 0)),
            pl.BlockSpec((1, D), lambda i: (0, 0)),
            pl.BlockSpec((NCLS, D), lambda i: (0, 0)),
            pl.BlockSpec((1, NCLS), lambda i: (0, 0)),
        ],
        out_specs=pl.BlockSpec((blk, NCLS), lambda i: (i, 0)),
        out_shape=jax.ShapeDtypeStruct((n_edges, NCLS), jnp.float32),
    )(g, st, pa0, pb0, gamma_row, beta_row, w2, b2_row)


def kernel(edge_index, embed, hyperedge_chunk_sizes, W1, b1, gamma, beta,
           W2, b2):
    e = edge_index.shape[1]
    src = edge_index[0]
    snk = edge_index[1]

    n_chunks = -(-e // (NW * CHUNK))
    n_chunks += (-n_chunks) % 3
    e_pad = n_chunks * CHUNK * NW
    if e_pad != e:
        pad = e_pad - e
        src = jnp.concatenate([src, jnp.zeros((pad,), jnp.int32)])
        snk = jnp.concatenate([snk, jnp.zeros((pad,), jnp.int32)])

    wa = W1[:, :D].T
    wb = W1[:, D:].T
    pa, pb = _node_tables(embed, wa, wb, b1.reshape(1, D))

    g, st = _gather_add(src, snk, pa, pb, e_pad)

    out = _fc2(g, st, pa[0:1], pb[0:1], gamma.reshape(1, D),
               beta.reshape(1, D), W2, b2.reshape(1, NCLS), e, e_pad - e)
    return out
